# baseline (speedup 1.0000x reference)
import jax
import jax.numpy as jnp
from jax.experimental import pallas as pl
from jax.experimental.pallas import tpu as pltpu

H = 16
KB = 2


def _msv_kernel(x_ref, wd_ref, wv_ref, bd_ref, bv_ref, t_ref, macc, vacc):
    kb = pl.program_id(1)
    xb = x_ref[...]
    dn = (((1,), (1,)), ((), ()))
    dm = jax.lax.dot_general(xb, wd_ref[...], dn,
                             preferred_element_type=jnp.float32)
    dv = jax.lax.dot_general(xb, wv_ref[0], dn,
                             preferred_element_type=jnp.float32)

    @pl.when(kb == 0)
    def _():
        macc[...] = dm
        vacc[...] = dv

    @pl.when(kb > 0)
    def _():
        macc[...] += dm
        vacc[...] += dv

    @pl.when(kb == KB - 1)
    def _():
        m = macc[...] + bd_ref[0]
        v = vacc[...] + bv_ref[0, 0]
        dh = m.shape[-1] // H
        for h in range(H):
            sl = slice(h * dh, (h + 1) * dh)
            mh = m[:, sl]
            mh = mh - jnp.max(mh, axis=-1, keepdims=True)
            e = jnp.exp(mh)
            t_ref[:, sl] = ((e / jnp.sum(e, axis=-1, keepdims=True))
                            * v[:, sl]).astype(jnp.bfloat16)


def _proj_kernel(t_ref, w_ref, b_ref, x_ref, o_ref):
    dn = (((1,), (1,)), ((), ()))
    o_ref[...] = (x_ref[...]
                  + jax.lax.dot_general(t_ref[...].astype(jnp.float32), w_ref[...],
                                        dn, preferred_element_type=jnp.float32)
                  + b_ref[0])


def kernel(x, Wqkv, bqkv, Wproj, bproj):
    B, S, D = x.shape
    dh = D // H
    scale = dh ** -0.5
    x2 = x.reshape(S, D)
    Wd = (Wqkv[:D] - Wqkv[D:2 * D]) * scale
    bd = ((bqkv[:D] - bqkv[D:2 * D]) * scale).reshape(1, D)
    w3 = Wqkv.reshape(3, D, D)
    b3 = bqkv.reshape(3, 1, D)

    BS1 = 512
    KH = D // KB
    t = pl.pallas_call(
        _msv_kernel,
        grid=(S // BS1, KB),
        in_specs=[
            pl.BlockSpec((BS1, KH), lambda i, kb: (i, kb)),
            pl.BlockSpec((D, KH), lambda i, kb: (0, kb)),
            pl.BlockSpec((1, D, KH), lambda i, kb: (2, 0, kb)),
            pl.BlockSpec((1, D), lambda i, kb: (0, 0)),
            pl.BlockSpec((1, 1, D), lambda i, kb: (2, 0, 0)),
        ],
        out_specs=pl.BlockSpec((BS1, D), lambda i, kb: (i, 0)),
        out_shape=jax.ShapeDtypeStruct((S, D), jnp.bfloat16),
        scratch_shapes=[pltpu.VMEM((BS1, D), jnp.float32),
                        pltpu.VMEM((BS1, D), jnp.float32)],
    )(x2, Wd, w3, bd, b3)

    BS2 = 512
    out = pl.pallas_call(
        _proj_kernel,
        grid=(S // BS2,),
        in_specs=[
            pl.BlockSpec((BS2, D), lambda i: (i, 0)),
            pl.BlockSpec((D, D), lambda i: (0, 0)),
            pl.BlockSpec((1, D), lambda i: (0, 0)),
            pl.BlockSpec((BS2, D), lambda i: (i, 0)),
        ],
        out_specs=pl.BlockSpec((BS2, D), lambda i: (i, 0)),
        out_shape=jax.ShapeDtypeStruct((S, D), jnp.float32),
    )(t, Wproj, bproj.reshape(1, D), x2)

    return out.reshape(B, S, D)


# speedup vs baseline: 1.3424x; 1.3424x over previous
import jax
import jax.numpy as jnp
from jax.experimental import pallas as pl
from jax.experimental.pallas import tpu as pltpu

H = 16


def _fused_kernel(x_ref, wd_ref, w3_ref, wp_ref, bd_ref, b3_ref, bp_ref,
                  o_ref, t_ref):
    xb = x_ref[...]
    dn = (((1,), (1,)), ((), ()))
    m = jax.lax.dot_general(xb, wd_ref[...], dn,
                            preferred_element_type=jnp.float32) + bd_ref[0]
    v = jax.lax.dot_general(xb, w3_ref[0], dn,
                            preferred_element_type=jnp.float32) + b3_ref[0, 0]
    dh = m.shape[-1] // H
    for h in range(H):
        sl = slice(h * dh, (h + 1) * dh)
        mh = m[:, sl]
        mh = mh - jnp.max(mh, axis=-1, keepdims=True)
        e = jnp.exp(mh)
        t_ref[:, sl] = (e / jnp.sum(e, axis=-1, keepdims=True)) * v[:, sl]
    o_ref[...] = (xb
                  + jax.lax.dot_general(t_ref[...], wp_ref[...], dn,
                                        preferred_element_type=jnp.float32)
                  + bp_ref[0])


def kernel(x, Wqkv, bqkv, Wproj, bproj):
    B, S, D = x.shape
    dh = D // H
    scale = dh ** -0.5
    x2 = x.reshape(S, D)
    Wd = (Wqkv[:D] - Wqkv[D:2 * D]) * scale
    bd = ((bqkv[:D] - bqkv[D:2 * D]) * scale).reshape(1, D)
    w3 = Wqkv.reshape(3, D, D)
    b3 = bqkv.reshape(3, 1, D)

    BS = 256
    out = pl.pallas_call(
        _fused_kernel,
        grid=(S // BS,),
        in_specs=[
            pl.BlockSpec((BS, D), lambda i: (i, 0)),
            pl.BlockSpec((D, D), lambda i: (0, 0)),
            pl.BlockSpec((1, D, D), lambda i: (2, 0, 0)),
            pl.BlockSpec((D, D), lambda i: (0, 0)),
            pl.BlockSpec((1, D), lambda i: (0, 0)),
            pl.BlockSpec((1, 1, D), lambda i: (2, 0, 0)),
            pl.BlockSpec((1, D), lambda i: (0, 0)),
        ],
        out_specs=pl.BlockSpec((BS, D), lambda i: (i, 0)),
        out_shape=jax.ShapeDtypeStruct((S, D), jnp.float32),
        scratch_shapes=[pltpu.VMEM((BS, D), jnp.float32)],
        compiler_params=pltpu.CompilerParams(vmem_limit_bytes=63 * 1024 * 1024),
    )(x2, Wd, w3, Wproj, bd, b3, bproj.reshape(1, D))

    return out.reshape(B, S, D)


# speedup vs baseline: 1.3852x; 1.0319x over previous
import jax
import jax.numpy as jnp
from jax.experimental import pallas as pl
from jax.experimental.pallas import tpu as pltpu

H = 16


def _fused_kernel(x_ref, wd_ref, w3_ref, wp_ref, bd_ref, b3_ref, bp_ref,
                  o_ref, t_ref):
    xb = x_ref[...]
    dn = (((1,), (1,)), ((), ()))
    m = jax.lax.dot_general(xb.astype(jnp.bfloat16), wd_ref[...], dn,
                            preferred_element_type=jnp.float32) + bd_ref[0]
    v = jax.lax.dot_general(xb, w3_ref[0], dn,
                            preferred_element_type=jnp.float32) + b3_ref[0, 0]
    dh = m.shape[-1] // H
    for h in range(H):
        sl = slice(h * dh, (h + 1) * dh)
        mh = m[:, sl]
        mh = mh - jnp.max(mh, axis=-1, keepdims=True)
        e = jnp.exp(mh)
        t_ref[:, sl] = (e / jnp.sum(e, axis=-1, keepdims=True)) * v[:, sl]
    o_ref[...] = (xb
                  + jax.lax.dot_general(t_ref[...], wp_ref[...], dn,
                                        preferred_element_type=jnp.float32)
                  + bp_ref[0])


def kernel(x, Wqkv, bqkv, Wproj, bproj):
    B, S, D = x.shape
    dh = D // H
    scale = dh ** -0.5
    x2 = x.reshape(S, D)
    Wd = ((Wqkv[:D] - Wqkv[D:2 * D]) * scale).astype(jnp.bfloat16)
    bd = ((bqkv[:D] - bqkv[D:2 * D]) * scale).reshape(1, D)
    w3 = Wqkv.reshape(3, D, D)
    b3 = bqkv.reshape(3, 1, D)

    BS = 256
    out = pl.pallas_call(
        _fused_kernel,
        grid=(S // BS,),
        in_specs=[
            pl.BlockSpec((BS, D), lambda i: (i, 0)),
            pl.BlockSpec((D, D), lambda i: (0, 0)),
            pl.BlockSpec((1, D, D), lambda i: (2, 0, 0)),
            pl.BlockSpec((D, D), lambda i: (0, 0)),
            pl.BlockSpec((1, D), lambda i: (0, 0)),
            pl.BlockSpec((1, 1, D), lambda i: (2, 0, 0)),
            pl.BlockSpec((1, D), lambda i: (0, 0)),
        ],
        out_specs=pl.BlockSpec((BS, D), lambda i: (i, 0)),
        out_shape=jax.ShapeDtypeStruct((S, D), jnp.float32),
        scratch_shapes=[pltpu.VMEM((BS, D), jnp.float32)],
        compiler_params=pltpu.CompilerParams(vmem_limit_bytes=63 * 1024 * 1024),
    )(x2, Wd, w3, Wproj, bd, b3, bproj.reshape(1, D))

    return out.reshape(B, S, D)


# speedup vs baseline: 1.4007x; 1.0112x over previous
import functools

import jax
import jax.numpy as jnp
from jax.experimental import pallas as pl
from jax.experimental.pallas import tpu as pltpu

H = 16


def _fused_kernel(x_ref, wd_ref, w3_ref, wp_ref, b3_ref, bp_ref,
                  o_ref, t_ref, *, scale):
    xb = x_ref[...]
    dn = (((1,), (1,)), ((), ()))
    m = (jax.lax.dot_general(xb.astype(jnp.bfloat16), wd_ref[...], dn,
                             preferred_element_type=jnp.float32)
         + (b3_ref[0, 0] - b3_ref[1, 0]) * scale)
    v = jax.lax.dot_general(xb, w3_ref[0], dn,
                            preferred_element_type=jnp.float32) + b3_ref[2, 0]
    dh = m.shape[-1] // H
    for h in range(H):
        sl = slice(h * dh, (h + 1) * dh)
        mh = m[:, sl]
        mh = mh - jnp.max(mh, axis=-1, keepdims=True)
        e = jnp.exp(mh)
        t_ref[:, sl] = (e / jnp.sum(e, axis=-1, keepdims=True)) * v[:, sl]
    o_ref[...] = (xb
                  + jax.lax.dot_general(t_ref[...], wp_ref[...], dn,
                                        preferred_element_type=jnp.float32)
                  + bp_ref[0])


def kernel(x, Wqkv, bqkv, Wproj, bproj):
    B, S, D = x.shape
    dh = D // H
    scale = dh ** -0.5
    x2 = x.reshape(S, D)
    Wd = ((Wqkv[:D] - Wqkv[D:2 * D]) * scale).astype(jnp.bfloat16)
    w3 = Wqkv.reshape(3, D, D)
    b3 = bqkv.reshape(3, 1, D)

    BS = 256
    out = pl.pallas_call(
        functools.partial(_fused_kernel, scale=scale),
        grid=(S // BS,),
        in_specs=[
            pl.BlockSpec((BS, D), lambda i: (i, 0)),
            pl.BlockSpec((D, D), lambda i: (0, 0)),
            pl.BlockSpec((1, D, D), lambda i: (2, 0, 0)),
            pl.BlockSpec((D, D), lambda i: (0, 0)),
            pl.BlockSpec((3, 1, D), lambda i: (0, 0, 0)),
            pl.BlockSpec((1, D), lambda i: (0, 0)),
        ],
        out_specs=pl.BlockSpec((BS, D), lambda i: (i, 0)),
        out_shape=jax.ShapeDtypeStruct((S, D), jnp.float32),
        scratch_shapes=[pltpu.VMEM((BS, D), jnp.float32)],
        compiler_params=pltpu.CompilerParams(vmem_limit_bytes=63 * 1024 * 1024),
    )(x2, Wd, w3, Wproj, b3, bproj.reshape(1, D))

    return out.reshape(B, S, D)


# speedup vs baseline: 1.4012x; 1.0004x over previous
import functools

import jax
import jax.numpy as jnp
from jax.experimental import pallas as pl
from jax.experimental.pallas import tpu as pltpu

H = 16


def _fused_kernel(x_ref, wd_ref, w3_ref, wp_ref, b3_ref, bp_ref,
                  o_ref, t_ref, *, scale):
    xb = x_ref[...]
    dn = (((1,), (1,)), ((), ()))
    m = (jax.lax.dot_general(xb.astype(jnp.bfloat16), wd_ref[...], dn,
                             preferred_element_type=jnp.float32)
         + (b3_ref[0, 0] - b3_ref[1, 0]) * scale)
    v = jax.lax.dot_general(xb, w3_ref[0], dn,
                            preferred_element_type=jnp.float32) + b3_ref[2, 0]
    dh = m.shape[-1] // H
    for h in range(H):
        sl = slice(h * dh, (h + 1) * dh)
        mh = m[:, sl]
        mh = mh - jnp.max(mh, axis=-1, keepdims=True)
        e = jnp.exp(mh)
        t_ref[:, sl] = (e / jnp.sum(e, axis=-1, keepdims=True)) * v[:, sl]
    o_ref[...] = (xb
                  + jax.lax.dot_general(t_ref[...], wp_ref[...], dn,
                                        preferred_element_type=jnp.float32)
                  + bp_ref[0])


def kernel(x, Wqkv, bqkv, Wproj, bproj):
    B, S, D = x.shape
    dh = D // H
    scale = dh ** -0.5
    x2 = x.reshape(S, D)
    Wd = ((Wqkv[:D] - Wqkv[D:2 * D]) * scale).astype(jnp.bfloat16)
    w3 = Wqkv.reshape(3, D, D)
    b3 = bqkv.reshape(3, 1, D)

    BS = 256
    out = pl.pallas_call(
        functools.partial(_fused_kernel, scale=scale),
        grid=(S // BS,),
        in_specs=[
            pl.BlockSpec((BS, D), lambda i: (i, 0)),
            pl.BlockSpec((D, D), lambda i: (0, 0)),
            pl.BlockSpec((1, D, D), lambda i: (2, 0, 0)),
            pl.BlockSpec((D, D), lambda i: (0, 0)),
            pl.BlockSpec((3, 1, D), lambda i: (0, 0, 0)),
            pl.BlockSpec((1, D), lambda i: (0, 0)),
        ],
        out_specs=pl.BlockSpec((BS, D), lambda i: (i, 0)),
        out_shape=jax.ShapeDtypeStruct((S, D), jnp.float32),
        scratch_shapes=[pltpu.VMEM((BS, D), jnp.float32)],
        compiler_params=pltpu.CompilerParams(
            vmem_limit_bytes=63 * 1024 * 1024,
            allow_input_fusion=[False, True, False, False, False, False],
        ),
    )(x2, Wd, w3, Wproj, b3, bproj.reshape(1, D))

    return out.reshape(B, S, D)
